# baseline (speedup 1.0000x reference)
import jax
import jax.numpy as jnp
from jax import lax
from jax.experimental import pallas as pl
from jax.experimental.pallas import tpu as pltpu

_TBLK = 8
_NBUF = 8


def _body(table_ref, x_hbm, o_hbm, inb, outb, insem, outsem):
    B, T, N, C = x_hbm.shape
    TT = T // _TBLK
    NCH = B * TT

    def in_copy(i, slot):
        b = i // TT
        t0 = (i % TT) * _TBLK
        return pltpu.make_async_copy(
            x_hbm.at[b, pl.ds(t0, _TBLK)], inb.at[slot], insem.at[slot]
        )

    def out_copy(i, slot):
        b = i // TT
        t0 = (i % TT) * _TBLK
        return pltpu.make_async_copy(
            outb.at[slot], o_hbm.at[b, pl.ds(t0, _TBLK)], outsem.at[slot]
        )

    for i in range(_NBUF):
        in_copy(i, i).start()

    def step(i, _):
        slot = i % _NBUF

        @pl.when(i >= _NBUF)
        def _():
            out_copy(i - _NBUF, slot).wait()

        in_copy(i, slot).wait()
        t0 = (i % TT) * _TBLK
        emb = table_ref[pl.ds(t0, _TBLK), :]
        outb[slot] = inb[slot] + emb[:, None, :]
        out_copy(i, slot).start()

        @pl.when(i + _NBUF < NCH)
        def _():
            in_copy(i + _NBUF, slot).start()

        return 0

    lax.fori_loop(0, NCH, step, 0)

    for i in range(NCH - _NBUF, NCH):
        out_copy(i, i % _NBUF).wait()


def kernel(x, table):
    B, T, N, C = x.shape
    R = N * C // 128
    x2 = x.reshape(B, T, R, 128)
    table2 = jnp.concatenate([table, table], axis=1)

    out2 = pl.pallas_call(
        _body,
        in_specs=[
            pl.BlockSpec(memory_space=pltpu.VMEM),
            pl.BlockSpec(memory_space=pl.ANY),
        ],
        out_specs=pl.BlockSpec(memory_space=pl.ANY),
        out_shape=jax.ShapeDtypeStruct((B, T, R, 128), x.dtype),
        scratch_shapes=[
            pltpu.VMEM((_NBUF, _TBLK, R, 128), x.dtype),
            pltpu.VMEM((_NBUF, _TBLK, R, 128), x.dtype),
            pltpu.SemaphoreType.DMA((_NBUF,)),
            pltpu.SemaphoreType.DMA((_NBUF,)),
        ],
    )(table2, x2)
    return out2.reshape(B, T, N, C)
